# Initial kernel scaffold; baseline (speedup 1.0000x reference)
import jax
import jax.numpy as jnp
from jax.experimental import pallas as pl


def kernel(x, emb, W):
    raise NotImplementedError("write your pallas kernel here")


# speedup vs baseline: 7.9181x; 7.9181x over previous
import functools

import jax
import jax.numpy as jnp
from jax import lax
from jax.experimental import pallas as pl
from jax.experimental.pallas import tpu as pltpu
from jax.experimental.pallas import tpu_sc as plsc

NC = 2
NS = 16
NW = NC * NS
CHUNK = 128


def _make_gather(n_rows, d):
  rows_per_w = n_rows // NW
  n_chunks = rows_per_w // CHUNK
  mesh = plsc.VectorSubcoreMesh(core_axis_name="c", subcore_axis_name="s")

  @functools.partial(
      pl.kernel,
      out_type=jax.ShapeDtypeStruct((n_rows, d), jnp.float32),
      mesh=mesh,
      scratch_types=[
          pltpu.VMEM((n_chunks, CHUNK), jnp.int32),
          pltpu.VMEM((CHUNK, d), jnp.float32),
          pltpu.SemaphoreType.DMA,
      ],
      compiler_params=pltpu.CompilerParams(use_tc_tiling_on_sc=False),
  )
  def gather(table_hbm, idx_hbm, out_hbm, idx_v, rows_v, sem):
    wid = lax.axis_index("s") * NC + lax.axis_index("c")
    base = wid * rows_per_w
    pltpu.sync_copy(idx_hbm.at[wid], idx_v)

    def body(c, _):
      pltpu.async_copy(table_hbm.at[idx_v.at[c]], rows_v, sem).wait()
      pltpu.sync_copy(rows_v, out_hbm.at[pl.ds(base + c * CHUNK, CHUNK)])
      return ()

    lax.fori_loop(0, n_chunks, body, (), unroll=False)

  return gather


def _project(e, w):
  n, k = e.shape
  m = w.shape[0]
  blk = 2048
  grid = n // blk

  def body(e_ref, w_ref, o_ref):
    o_ref[...] = lax.dot_general(
        e_ref[...], w_ref[...], (((1,), (1,)), ((), ())),
        preferred_element_type=jnp.float32)

  return pl.pallas_call(
      body,
      grid=(grid,),
      in_specs=[
          pl.BlockSpec((blk, k), lambda i: (i, 0)),
          pl.BlockSpec((m, k), lambda i: (0, 0)),
      ],
      out_specs=pl.BlockSpec((blk, m), lambda i: (i, 0)),
      out_shape=jax.ShapeDtypeStruct((n, m), jnp.float32),
  )(e, w)


@jax.jit
def kernel(x, emb, W):
  b, f = x.shape
  n = b * f
  d = emb.shape[1]
  idx = x.reshape(NW, n // NW // CHUNK, CHUNK)
  e = _make_gather(n, d)(emb, idx)
  out = _project(e, W)
  return out.reshape(b, f, W.shape[0])


# speedup vs baseline: 15.4378x; 1.9497x over previous
import functools

import jax
import jax.numpy as jnp
from jax import lax
from jax.experimental import pallas as pl
from jax.experimental.pallas import tpu as pltpu
from jax.experimental.pallas import tpu_sc as plsc

NC = 2
NS = 16
NW = NC * NS
CHUNK = 128
PCOL = 4
BLK = 2048


def _make_gather(n_rows, d):
  rows_per_w = n_rows // NW
  n_chunks = rows_per_w // CHUNK
  wg = NW // PCOL
  mesh = plsc.VectorSubcoreMesh(core_axis_name="c", subcore_axis_name="s")

  @functools.partial(
      pl.kernel,
      out_type=jax.ShapeDtypeStruct((n_rows // PCOL, PCOL * d), jnp.float32),
      mesh=mesh,
      scratch_types=[
          pltpu.VMEM((n_chunks, CHUNK), jnp.int32),
          pltpu.VMEM((CHUNK, d), jnp.float32),
          pltpu.SemaphoreType.DMA,
      ],
      compiler_params=pltpu.CompilerParams(use_tc_tiling_on_sc=False),
  )
  def gather(table_hbm, idx_hbm, out_hbm, idx_v, rows_v, sem):
    wid = lax.axis_index("s") * NC + lax.axis_index("c")
    p = wid // wg
    r0 = (wid % wg) * rows_per_w
    pltpu.sync_copy(idx_hbm.at[wid], idx_v)

    def body(c, _):
      pltpu.async_copy(table_hbm.at[idx_v.at[c]], rows_v, sem).wait()
      pltpu.sync_copy(
          rows_v,
          out_hbm.at[pl.ds(r0 + c * CHUNK, CHUNK), pl.ds(p * d, d)])
      return ()

    lax.fori_loop(0, n_chunks, body, (), unroll=False)

  return gather


def _project(e4, wt):
  n4, lanes = e4.shape
  d = lanes // PCOL
  m = wt.shape[1]
  grid = n4 // BLK

  def body(e_ref, w_ref, o_ref):
    for p in range(PCOL):
      o_ref[p, :, :] = lax.dot_general(
          e_ref[:, p * d:(p + 1) * d], w_ref[...],
          (((1,), (0,)), ((), ())), preferred_element_type=jnp.float32)

  return pl.pallas_call(
      body,
      grid=(grid,),
      in_specs=[
          pl.BlockSpec((BLK, lanes), lambda i: (i, 0)),
          pl.BlockSpec((d, m), lambda i: (0, 0)),
      ],
      out_specs=pl.BlockSpec((PCOL, BLK, m), lambda i: (0, i, 0)),
      out_shape=jax.ShapeDtypeStruct((PCOL, n4, m), jnp.float32),
      compiler_params=pltpu.CompilerParams(
          dimension_semantics=("arbitrary",)),
  )(e4, wt)


@jax.jit
def kernel(x, emb, W):
  b, f = x.shape
  n = b * f
  d = emb.shape[1]
  m = W.shape[0]
  idx = x.T.reshape(NW, n // NW // CHUNK, CHUNK)
  e4 = _make_gather(n, d)(emb, idx)
  out4 = _project(e4, W.T)
  return out4.reshape(f, b, m).transpose(1, 0, 2)


# speedup vs baseline: 22.9590x; 1.4872x over previous
import functools

import jax
import jax.numpy as jnp
from jax import lax
from jax.experimental import pallas as pl
from jax.experimental.pallas import tpu as pltpu
from jax.experimental.pallas import tpu_sc as plsc

NC = 2
NS = 16
NW = NC * NS
CHUNK = 128
PCOL = 4
BLK = 2048


def _make_gather(n_rows, d):
  rows_per_w = n_rows // NW
  n_chunks = rows_per_w // CHUNK
  wg = NW // PCOL
  mesh = plsc.VectorSubcoreMesh(core_axis_name="c", subcore_axis_name="s")

  @functools.partial(
      pl.kernel,
      out_type=jax.ShapeDtypeStruct((n_rows // PCOL, PCOL * d), jnp.float32),
      mesh=mesh,
      scratch_types=[
          pltpu.VMEM((n_chunks, CHUNK), jnp.int32),
          pltpu.VMEM((CHUNK, d), jnp.float32),
          pltpu.SemaphoreType.DMA,
      ],
      compiler_params=pltpu.CompilerParams(use_tc_tiling_on_sc=False),
  )
  def gather(table_hbm, idx_hbm, out_hbm, idx_v, rows_v, sem):
    wid = lax.axis_index("s") * NC + lax.axis_index("c")
    p = wid // wg
    r0 = (wid % wg) * rows_per_w
    pltpu.sync_copy(idx_hbm.at[wid], idx_v)

    def body(c, _):
      pltpu.async_copy(table_hbm.at[idx_v.at[c]], rows_v, sem).wait()
      pltpu.sync_copy(
          rows_v,
          out_hbm.at[pl.ds(r0 + c * CHUNK, CHUNK), pl.ds(p * d, d)])
      return ()

    lax.fori_loop(0, n_chunks, body, (), unroll=False)

  return gather


CB = 8192
SUB = CB // PCOL


def _to_packed(embT):
  d, n = embT.shape
  grid = pl.cdiv(n, CB)

  def body(in_ref, o_ref):
    eye = jnp.eye(d, dtype=jnp.float32)
    for a in range(PCOL):
      o_ref[:, d * a:d * (a + 1)] = lax.dot_general(
          in_ref[:, a * SUB:(a + 1) * SUB], eye, (((0,), (0,)), ((), ())),
          preferred_element_type=jnp.float32)

  return pl.pallas_call(
      body,
      grid=(grid,),
      in_specs=[pl.BlockSpec((d, CB), lambda i: (0, i))],
      out_specs=pl.BlockSpec((SUB, PCOL * d), lambda i: (i, 0)),
      out_shape=jax.ShapeDtypeStruct((grid * SUB, PCOL * d), jnp.float32),
      compiler_params=pltpu.CompilerParams(
          dimension_semantics=("arbitrary",)),
  )(embT)


def _project(e4, wt):
  n4, lanes = e4.shape
  d = lanes // PCOL
  m = wt.shape[1]
  grid = n4 // BLK

  def body(e_ref, w_ref, o_ref):
    for p in range(PCOL):
      o_ref[p, :, :] = lax.dot_general(
          e_ref[:, p * d:(p + 1) * d], w_ref[...],
          (((1,), (0,)), ((), ())), preferred_element_type=jnp.float32)

  return pl.pallas_call(
      body,
      grid=(grid,),
      in_specs=[
          pl.BlockSpec((BLK, lanes), lambda i: (i, 0)),
          pl.BlockSpec((d, m), lambda i: (0, 0)),
      ],
      out_specs=pl.BlockSpec((PCOL, BLK, m), lambda i: (0, i, 0)),
      out_shape=jax.ShapeDtypeStruct((PCOL, n4, m), jnp.float32),
      compiler_params=pltpu.CompilerParams(
          dimension_semantics=("arbitrary",)),
  )(e4, wt)


@jax.jit
def kernel(x, emb, W):
  b, f = x.shape
  n = b * f
  d = emb.shape[1]
  m = W.shape[0]
  xt = x.T
  jdx = ((xt // CB) * SUB + xt % SUB) * PCOL + (xt % CB) // SUB
  idx = jdx.reshape(NW, n // NW // CHUNK, CHUNK)
  packed = _to_packed(emb.T)
  table = packed.reshape(packed.shape[0] * PCOL, d)
  e4 = _make_gather(n, d)(table, idx)
  out4 = _project(e4, W.T)
  return out4.reshape(f, b, m).transpose(1, 0, 2)


# speedup vs baseline: 24.3503x; 1.0606x over previous
import functools

import jax
import jax.numpy as jnp
from jax import lax
from jax.experimental import pallas as pl
from jax.experimental.pallas import tpu as pltpu
from jax.experimental.pallas import tpu_sc as plsc

NC = 2
NS = 16
NW = NC * NS
CHUNK = 128
PCOL = 4
BLK = 2048


def _make_gather(n_rows, d):
  rows_per_w = n_rows // NW
  n_chunks = rows_per_w // CHUNK
  wg = NW // PCOL
  mesh = plsc.VectorSubcoreMesh(core_axis_name="c", subcore_axis_name="s")

  @functools.partial(
      pl.kernel,
      out_type=jax.ShapeDtypeStruct((n_rows // PCOL, PCOL * d), jnp.float32),
      mesh=mesh,
      scratch_types=[
          pltpu.VMEM((n_chunks, CHUNK), jnp.int32),
          pltpu.VMEM((CHUNK, d), jnp.float32),
          pltpu.SemaphoreType.DMA,
      ],
      compiler_params=pltpu.CompilerParams(use_tc_tiling_on_sc=False),
  )
  def gather(table_hbm, idx_hbm, out_hbm, idx_v, rows_v, sem):
    wid = lax.axis_index("s") * NC + lax.axis_index("c")
    p = wid // wg
    r0 = (wid % wg) * rows_per_w
    pltpu.sync_copy(idx_hbm.at[wid], idx_v)

    def body(c, _):
      pltpu.async_copy(table_hbm.at[idx_v.at[c]], rows_v, sem).wait()
      pltpu.sync_copy(
          rows_v,
          out_hbm.at[pl.ds(r0 + c * CHUNK, CHUNK), pl.ds(p * d, d)])
      return ()

    lax.fori_loop(0, n_chunks, body, (), unroll=False)

  return gather


CB = 8192
SUB = CB // PCOL


def _to_packed(embT):
  d, n = embT.shape
  grid = pl.cdiv(n, CB)

  def body(in_ref, o_ref):
    t = in_ref[...].T
    for a in range(PCOL):
      o_ref[:, d * a:d * (a + 1)] = t[a * SUB:(a + 1) * SUB, :]

  return pl.pallas_call(
      body,
      grid=(grid,),
      in_specs=[pl.BlockSpec((d, CB), lambda i: (0, i))],
      out_specs=pl.BlockSpec((SUB, PCOL * d), lambda i: (i, 0)),
      out_shape=jax.ShapeDtypeStruct((grid * SUB, PCOL * d), jnp.float32),
      compiler_params=pltpu.CompilerParams(
          dimension_semantics=("arbitrary",)),
  )(embT)


def _project_slab(e4, wt, out_prev, q, nch):
  n4, lanes = e4.shape
  d = lanes // PCOL
  m = wt.shape[1]
  grid = n4 // BLK

  def body(*refs):
    e_ref, w_ref = refs[0], refs[1]
    o_ref = refs[-1]
    for p in range(PCOL):
      o_ref[0, p, :, :] = lax.dot_general(
          e_ref[:, p * d:(p + 1) * d], w_ref[...],
          (((1,), (0,)), ((), ())), preferred_element_type=jnp.float32)

  in_specs = [
      pl.BlockSpec((BLK, lanes), lambda i: (i, 0)),
      pl.BlockSpec((d, m), lambda i: (0, 0)),
  ]
  args = [e4, wt]
  aliases = {}
  if out_prev is not None:
    in_specs.append(pl.BlockSpec(memory_space=pl.ANY))
    args.append(out_prev)
    aliases = {2: 0}
  return pl.pallas_call(
      body,
      grid=(grid,),
      in_specs=in_specs,
      out_specs=pl.BlockSpec((1, PCOL, BLK, m), lambda i, q=q: (q, 0, i, 0)),
      out_shape=jax.ShapeDtypeStruct((nch, PCOL, n4, m), jnp.float32),
      input_output_aliases=aliases,
      compiler_params=pltpu.CompilerParams(
          dimension_semantics=("arbitrary",)),
  )(*args)


NCH = 4


@jax.jit
def kernel(x, emb, W):
  b, f = x.shape
  n = b * f
  d = emb.shape[1]
  m = W.shape[0]
  xt = x.T
  jdx = (((xt // CB) * SUB + xt % SUB) * PCOL + (xt % CB) // SUB).reshape(n)
  packed = _to_packed(emb.T)
  table = packed.reshape(packed.shape[0] * PCOL, d)
  nq = n // NCH
  gather_fn = _make_gather(nq, d)
  wt = W.T
  out = None
  for q in range(NCH):
    idx_q = jdx[q * nq:(q + 1) * nq].reshape(NW, nq // NW // CHUNK, CHUNK)
    e4q = gather_fn(table, idx_q)
    out = _project_slab(e4q, wt, out, q, NCH)
  return out.reshape(f, b, m).transpose(1, 0, 2)
